# Initial kernel scaffold; baseline (speedup 1.0000x reference)
import functools

import jax
import jax.numpy as jnp
from jax import lax
from jax.experimental import pallas as pl
from jax.experimental.pallas import tpu as pltpu
from jax.experimental.pallas import tpu_sc as plsc

N = 65536
D = 128
B = 8192
NC = 2
NS = 16
L = 16
NW = NC * NS
ROWS_PER_W = N // NW
POS_PER_TILE = B // NS
CH = 128
NCH = POS_PER_TILE // CH
NK = D // L
BLK = 128
NBLK = ROWS_PER_W // BLK


def _body(x_hbm, idx0_hbm, idx1_hbm, out_hbm,
          idx0_v, idx1_v, ones_v, rows_v, xout_v, f0_v,
          partial_v, pall_v, zbuf_v, fl0_v, fl1_v,
          table0, table1, pspm, sem):
    c = lax.axis_index("c")
    s = lax.axis_index("s")
    wid = s * NC + c

    def zb(i, _):
        zbuf_v[pl.ds(i * L, L)] = jnp.zeros((L,), jnp.float32)
        return 0
    lax.fori_loop(0, (N // NS) // L, zb, 0)
    pltpu.sync_copy(zbuf_v, table0.at[pl.ds(s * (N // NS), N // NS)])
    pltpu.sync_copy(zbuf_v, table1.at[pl.ds(s * (N // NS), N // NS)])
    ones_v[...] = jnp.ones((CH,), jnp.float32)
    plsc.subcore_barrier()

    for j in range(NCH):
        pltpu.sync_copy(idx0_hbm.at[pl.ds(POS_PER_TILE * s + CH * j, CH)],
                        idx0_v.at[j])
        pltpu.sync_copy(idx1_hbm.at[pl.ds(POS_PER_TILE * s + CH * j, CH)],
                        idx1_v.at[j])
    for j in range(NCH):
        pltpu.sync_copy(ones_v, table0.at[idx0_v.at[j]], add=True)
        pltpu.sync_copy(ones_v, table1.at[idx1_v.at[j]], add=True)
    plsc.subcore_barrier()

    acc0 = [jnp.zeros((L,), jnp.float32)] * NK
    for j in range(NCH):
        pltpu.async_copy(x_hbm.at[idx0_v.at[j]], rows_v, sem).wait()

        def red0(i, carry):
            return tuple(carry[k] + rows_v[i, pl.ds(k * L, L)]
                         for k in range(NK))
        acc0 = list(lax.fori_loop(0, CH, red0, tuple(acc0)))

    accS = [jnp.zeros((L,), jnp.float32)] * NK
    accH = [jnp.zeros((L,), jnp.float32)] * NK
    accC = jnp.float32(0.0)
    for j in range(NCH):
        pltpu.async_copy(x_hbm.at[idx1_v.at[j]], rows_v, sem).wait()
        pltpu.sync_copy(table0.at[idx1_v.at[j]], f0_v)

        def red1(i, carry):
            sS, sH, sC = carry[:NK], carry[NK:2 * NK], carry[2 * NK]
            w = jnp.where(f0_v[i] > 0.0, jnp.float32(1.0), jnp.float32(0.0))
            wv = jnp.full((L,), w, jnp.float32)
            outS, outH = [], []
            for k in range(NK):
                v = rows_v[i, pl.ds(k * L, L)]
                outS.append(sS[k] + v)
                outH.append(sH[k] + wv * v)
            return tuple(outS) + tuple(outH) + (sC + w,)
        carry = lax.fori_loop(0, CH, red1, tuple(accS) + tuple(accH) + (accC,))
        accS, accH, accC = list(carry[:NK]), list(carry[NK:2 * NK]), carry[2 * NK]

    for k in range(NK):
        partial_v[pl.ds(k * L, L)] = acc0[k]
        partial_v[pl.ds(D + k * L, L)] = accS[k]
        partial_v[pl.ds(2 * D + k * L, L)] = accH[k]
    partial_v[pl.ds(3 * D, L)] = jnp.full((L,), accC, jnp.float32)
    partial_v[pl.ds(3 * D + L, L)] = jnp.zeros((L,), jnp.float32)
    pltpu.sync_copy(partial_v, pspm.at[s])
    plsc.subcore_barrier()
    pltpu.sync_copy(pspm, pall_v)

    def redp(t, carry):
        return tuple(carry[k] + pall_v[t, pl.ds(k * L, L)]
                     for k in range(3 * NK + 1))
    tot = lax.fori_loop(0, NS, redp,
                        tuple([jnp.zeros((L,), jnp.float32)] * (3 * NK + 1)))
    sum0 = tot[:NK]
    s1x = tot[NK:2 * NK]
    sh = tot[2 * NK:3 * NK]
    cv = tot[3 * NK]

    inv_b = jnp.float32(1.0 / B)
    m0 = [sum0[k] * inv_b for k in range(NK)]
    ilo8 = lax.iota(jnp.int32, L) < 8
    zero = jnp.zeros((L,), jnp.float32)
    m1 = []
    for k in range(NK):
        if k == 0:
            r = jnp.where(ilo8, sh[0], cv * m0[0])
        elif k == 1:
            r = jnp.where(ilo8, zero, cv * m0[1])
        else:
            r = cv * m0[k]
        m1.append((s1x[k] - sh[k] + r) * inv_b)
    rowz1 = jnp.where(ilo8, zero, m0[1])
    rowz2 = jnp.where(ilo8, zero, m1[1])

    base = wid * ROWS_PER_W
    pltpu.sync_copy(table0.at[pl.ds(base, ROWS_PER_W)], fl0_v)
    pltpu.sync_copy(table1.at[pl.ds(base, ROWS_PER_W)], fl1_v)

    def blk_body(b, _):
        r0 = base + b * BLK
        pltpu.sync_copy(x_hbm.at[pl.ds(r0, BLK)], rows_v)

        def srow(i, _2):
            g0 = fl0_v[b * BLK + i] > 0.0
            g1 = fl1_v[b * BLK + i] > 0.0
            for k in range(NK):
                v = rows_v[i, pl.ds(k * L, L)]
                if k == 0:
                    p1 = jnp.where(ilo8, v, m0[0])
                    p2 = jnp.where(ilo8, m1[0], jnp.where(g0, m0[0], v))
                elif k == 1:
                    p1 = rowz1
                    p2 = rowz2
                else:
                    p1 = m0[k]
                    p2 = m1[k]
                o = jnp.where(g1, p2, jnp.where(g0, p1, v))
                xout_v[i, pl.ds(k * L, L)] = o
            return 0
        lax.fori_loop(0, BLK, srow, 0)
        pltpu.sync_copy(xout_v, out_hbm.at[pl.ds(r0, BLK)])
        return 0
    lax.fori_loop(0, NBLK, blk_body, 0)


@jax.jit
def _selective_filter(x, idx0, idx1):
    mesh = plsc.VectorSubcoreMesh(core_axis_name="c", subcore_axis_name="s")
    return pl.kernel(
        _body,
        out_type=jax.ShapeDtypeStruct((N, D), jnp.float32),
        mesh=mesh,
        scratch_types=[
            pltpu.VMEM((NCH, CH), jnp.int32),
            pltpu.VMEM((NCH, CH), jnp.int32),
            pltpu.VMEM((CH,), jnp.float32),
            pltpu.VMEM((BLK, D), jnp.float32),
            pltpu.VMEM((BLK, D), jnp.float32),
            pltpu.VMEM((CH,), jnp.float32),
            pltpu.VMEM((512,), jnp.float32),
            pltpu.VMEM((NS, 512), jnp.float32),
            pltpu.VMEM((N // NS,), jnp.float32),
            pltpu.VMEM((ROWS_PER_W,), jnp.float32),
            pltpu.VMEM((ROWS_PER_W,), jnp.float32),
            pltpu.VMEM_SHARED((N,), jnp.float32),
            pltpu.VMEM_SHARED((N,), jnp.float32),
            pltpu.VMEM_SHARED((NS, 512), jnp.float32),
            pltpu.SemaphoreType.DMA,
        ],
    )(x, idx0, idx1)


def kernel(input, idx0, idx1):
    return _selective_filter(input, idx0, idx1)


# speedup vs baseline: 1.0276x; 1.0276x over previous
import functools

import jax
import jax.numpy as jnp
from jax import lax
from jax.experimental import pallas as pl
from jax.experimental.pallas import tpu as pltpu
from jax.experimental.pallas import tpu_sc as plsc

N = 65536
D = 128
B = 8192
NC = 2
NS = 16
L = 16
NW = NC * NS
ROWS_PER_W = N // NW
POS_PER_TILE = B // NS
CH = 128
NCH = POS_PER_TILE // CH
NK = D // L
BLK = 128
NBLK = ROWS_PER_W // BLK


def _body(x_hbm, idx0_hbm, idx1_hbm, out_hbm,
          idx0_v, idx1_v, ones_v, rows_v, xout_v, f0_v,
          partial_v, pall_v, zbuf_v, fl0_v, fl1_v,
          table0, table1, pspm, sem):
    c = lax.axis_index("c")
    s = lax.axis_index("s")
    wid = s * NC + c

    def zb(i, _):
        zbuf_v[pl.ds(i * L, L)] = jnp.zeros((L,), jnp.float32)
        return 0
    lax.fori_loop(0, (N // NS) // L, zb, 0)
    pltpu.sync_copy(zbuf_v, table0.at[pl.ds(s * (N // NS), N // NS)])
    pltpu.sync_copy(zbuf_v, table1.at[pl.ds(s * (N // NS), N // NS)])
    ones_v[...] = jnp.ones((CH,), jnp.float32)
    plsc.subcore_barrier()

    for j in range(NCH):
        pltpu.sync_copy(idx0_hbm.at[pl.ds(POS_PER_TILE * s + CH * j, CH)],
                        idx0_v.at[j])
        pltpu.sync_copy(idx1_hbm.at[pl.ds(POS_PER_TILE * s + CH * j, CH)],
                        idx1_v.at[j])
    for j in range(NCH):
        pltpu.sync_copy(ones_v, table0.at[idx0_v.at[j]], add=True)
        pltpu.sync_copy(ones_v, table1.at[idx1_v.at[j]], add=True)
    plsc.subcore_barrier()

    acc0 = [jnp.zeros((L,), jnp.float32)] * NK
    for j in range(NCH):
        pltpu.async_copy(x_hbm.at[idx0_v.at[j]], rows_v, sem).wait()

        def red0(i, carry):
            return tuple(carry[k] + rows_v[i, pl.ds(k * L, L)]
                         for k in range(NK))
        acc0 = list(lax.fori_loop(0, CH, red0, tuple(acc0)))

    accS = [jnp.zeros((L,), jnp.float32)] * NK
    accH = [jnp.zeros((L,), jnp.float32)] * NK
    accC = jnp.float32(0.0)
    for j in range(NCH):
        pltpu.async_copy(x_hbm.at[idx1_v.at[j]], rows_v, sem).wait()
        pltpu.sync_copy(table0.at[idx1_v.at[j]], f0_v.at[pl.ds(0, CH)])

        def red1(i, carry):
            sS, sH, sC = carry[:NK], carry[NK:2 * NK], carry[2 * NK]
            w = jnp.where(f0_v[pl.ds(i, L)][0] > 0.0,
                          jnp.float32(1.0), jnp.float32(0.0))
            wv = jnp.full((L,), w, jnp.float32)
            outS, outH = [], []
            for k in range(NK):
                v = rows_v[i, pl.ds(k * L, L)]
                outS.append(sS[k] + v)
                outH.append(sH[k] + wv * v)
            return tuple(outS) + tuple(outH) + (sC + w,)
        carry = lax.fori_loop(0, CH, red1, tuple(accS) + tuple(accH) + (accC,))
        accS, accH, accC = list(carry[:NK]), list(carry[NK:2 * NK]), carry[2 * NK]

    for k in range(NK):
        partial_v[pl.ds(k * L, L)] = acc0[k]
        partial_v[pl.ds(D + k * L, L)] = accS[k]
        partial_v[pl.ds(2 * D + k * L, L)] = accH[k]
    partial_v[pl.ds(3 * D, L)] = jnp.full((L,), accC, jnp.float32)
    partial_v[pl.ds(3 * D + L, L)] = jnp.zeros((L,), jnp.float32)
    pltpu.sync_copy(partial_v, pspm.at[s])
    plsc.subcore_barrier()
    pltpu.sync_copy(pspm, pall_v)

    def redp(t, carry):
        return tuple(carry[k] + pall_v[t, pl.ds(k * L, L)]
                     for k in range(3 * NK + 1))
    tot = lax.fori_loop(0, NS, redp,
                        tuple([jnp.zeros((L,), jnp.float32)] * (3 * NK + 1)))
    sum0 = tot[:NK]
    s1x = tot[NK:2 * NK]
    sh = tot[2 * NK:3 * NK]
    cv = tot[3 * NK]

    inv_b = jnp.float32(1.0 / B)
    m0 = [sum0[k] * inv_b for k in range(NK)]
    ilo8 = lax.iota(jnp.int32, L) < 8
    zero = jnp.zeros((L,), jnp.float32)
    m1 = []
    for k in range(NK):
        if k == 0:
            r = jnp.where(ilo8, sh[0], cv * m0[0])
        elif k == 1:
            r = jnp.where(ilo8, zero, cv * m0[1])
        else:
            r = cv * m0[k]
        m1.append((s1x[k] - sh[k] + r) * inv_b)
    rowz1 = jnp.where(ilo8, zero, m0[1])
    rowz2 = jnp.where(ilo8, zero, m1[1])

    base = wid * ROWS_PER_W
    pltpu.sync_copy(table0.at[pl.ds(base, ROWS_PER_W)],
                    fl0_v.at[pl.ds(0, ROWS_PER_W)])
    pltpu.sync_copy(table1.at[pl.ds(base, ROWS_PER_W)],
                    fl1_v.at[pl.ds(0, ROWS_PER_W)])

    def blk_body(b, _):
        r0 = base + b * BLK
        pltpu.sync_copy(x_hbm.at[pl.ds(r0, BLK)], rows_v)

        def srow(i, _2):
            g0 = fl0_v[pl.ds(b * BLK + i, L)][0] > 0.0
            g1 = fl1_v[pl.ds(b * BLK + i, L)][0] > 0.0
            for k in range(NK):
                v = rows_v[i, pl.ds(k * L, L)]
                if k == 0:
                    p1 = jnp.where(ilo8, v, m0[0])
                    p2 = jnp.where(ilo8, m1[0], jnp.where(g0, m0[0], v))
                elif k == 1:
                    p1 = rowz1
                    p2 = rowz2
                else:
                    p1 = m0[k]
                    p2 = m1[k]
                o = jnp.where(g1, p2, jnp.where(g0, p1, v))
                xout_v[i, pl.ds(k * L, L)] = o
            return 0
        lax.fori_loop(0, BLK, srow, 0)
        pltpu.sync_copy(xout_v, out_hbm.at[pl.ds(r0, BLK)])
        return 0
    lax.fori_loop(0, NBLK, blk_body, 0)


@jax.jit
def _selective_filter(x, idx0, idx1):
    mesh = plsc.VectorSubcoreMesh(core_axis_name="c", subcore_axis_name="s")
    return pl.kernel(
        _body,
        out_type=jax.ShapeDtypeStruct((N, D), jnp.float32),
        mesh=mesh,
        scratch_types=[
            pltpu.VMEM((NCH, CH), jnp.int32),
            pltpu.VMEM((NCH, CH), jnp.int32),
            pltpu.VMEM((CH,), jnp.float32),
            pltpu.VMEM((BLK, D), jnp.float32),
            pltpu.VMEM((BLK, D), jnp.float32),
            pltpu.VMEM((CH + L,), jnp.float32),
            pltpu.VMEM((512,), jnp.float32),
            pltpu.VMEM((NS, 512), jnp.float32),
            pltpu.VMEM((N // NS,), jnp.float32),
            pltpu.VMEM((ROWS_PER_W + L,), jnp.float32),
            pltpu.VMEM((ROWS_PER_W + L,), jnp.float32),
            pltpu.VMEM_SHARED((N,), jnp.float32),
            pltpu.VMEM_SHARED((N,), jnp.float32),
            pltpu.VMEM_SHARED((NS, 512), jnp.float32),
            pltpu.SemaphoreType.DMA,
        ],
    )(x, idx0, idx1)


def kernel(input, idx0, idx1):
    return _selective_filter(input, idx0, idx1)


# speedup vs baseline: 1.9223x; 1.8707x over previous
import jax
import jax.numpy as jnp
from jax import lax
from jax.experimental import pallas as pl
from jax.experimental.pallas import tpu as pltpu
from jax.experimental.pallas import tpu_sc as plsc

N = 65536
D = 128
B = 8192
NC = 2
NS = 16
L = 16
NW = NC * NS
ROWS_PER_W = N // NW
POS_PER_TILE = B // NS
CH = 128
NCH = POS_PER_TILE // CH
NK = D // L
LSZ = ROWS_PER_W + 2 * CH


def _body(x_hbm, idx0_hbm, idx1_hbm, out_hbm,
          idx0_v, idx1_v, ones_v, rows_v, f0_v,
          partial_v, pall_v, zbuf_v, fl0_v, fl1_v,
          lst_v, idxw_v,
          table0, table1, pspm, sem):
    c = lax.axis_index("c")
    s = lax.axis_index("s")
    wid = s * NC + c

    def zb(i, _):
        zbuf_v[pl.ds(i * L, L)] = jnp.zeros((L,), jnp.float32)
        return 0
    lax.fori_loop(0, (N // NS) // L, zb, 0)
    pltpu.sync_copy(zbuf_v, table0.at[pl.ds(s * (N // NS), N // NS)])
    pltpu.sync_copy(zbuf_v, table1.at[pl.ds(s * (N // NS), N // NS)])
    for k in range(CH // L):
        ones_v[pl.ds(k * L, L)] = jnp.ones((L,), jnp.float32)
    plsc.subcore_barrier()

    for j in range(NCH):
        pltpu.sync_copy(idx0_hbm.at[pl.ds(POS_PER_TILE * s + CH * j, CH)],
                        idx0_v.at[j])
        pltpu.sync_copy(idx1_hbm.at[pl.ds(POS_PER_TILE * s + CH * j, CH)],
                        idx1_v.at[j])
    for j in range(NCH):
        pltpu.sync_copy(ones_v, table0.at[idx0_v.at[j]], add=True)
        pltpu.sync_copy(ones_v, table1.at[idx1_v.at[j]], add=True)
    plsc.subcore_barrier()

    acc0 = [jnp.zeros((L,), jnp.float32)] * NK
    for j in range(NCH):
        pltpu.async_copy(x_hbm.at[idx0_v.at[j]], rows_v, sem).wait()

        def red0(i, carry):
            return tuple(carry[k] + rows_v[i, pl.ds(k * L, L)]
                         for k in range(NK))
        acc0 = list(lax.fori_loop(0, CH, red0, tuple(acc0)))

    accS = [jnp.zeros((L,), jnp.float32)] * NK
    accH = [jnp.zeros((L,), jnp.float32)] * NK
    accC = jnp.float32(0.0)
    for j in range(NCH):
        pltpu.async_copy(x_hbm.at[idx1_v.at[j]], rows_v, sem).wait()
        pltpu.sync_copy(table0.at[idx1_v.at[j]], f0_v.at[pl.ds(0, CH)])

        def red1(i, carry):
            sS, sH, sC = carry[:NK], carry[NK:2 * NK], carry[2 * NK]
            w = jnp.where(f0_v[pl.ds(i, L)][0] > 0.0,
                          jnp.float32(1.0), jnp.float32(0.0))
            wv = jnp.full((L,), w, jnp.float32)
            outS, outH = [], []
            for k in range(NK):
                v = rows_v[i, pl.ds(k * L, L)]
                outS.append(sS[k] + v)
                outH.append(sH[k] + wv * v)
            return tuple(outS) + tuple(outH) + (sC + w,)
        carry = lax.fori_loop(0, CH, red1, tuple(accS) + tuple(accH) + (accC,))
        accS, accH, accC = list(carry[:NK]), list(carry[NK:2 * NK]), carry[2 * NK]

    for k in range(NK):
        partial_v[pl.ds(k * L, L)] = acc0[k]
        partial_v[pl.ds(D + k * L, L)] = accS[k]
        partial_v[pl.ds(2 * D + k * L, L)] = accH[k]
    partial_v[pl.ds(3 * D, L)] = jnp.full((L,), accC, jnp.float32)
    partial_v[pl.ds(3 * D + L, L)] = jnp.zeros((L,), jnp.float32)
    pltpu.sync_copy(partial_v, pspm.at[s])
    plsc.subcore_barrier()
    pltpu.sync_copy(pspm, pall_v)

    def redp(t, carry):
        return tuple(carry[k] + pall_v[t, pl.ds(k * L, L)]
                     for k in range(3 * NK + 1))
    tot = lax.fori_loop(0, NS, redp,
                        tuple([jnp.zeros((L,), jnp.float32)] * (3 * NK + 1)))
    sum0 = tot[:NK]
    s1x = tot[NK:2 * NK]
    sh = tot[2 * NK:3 * NK]
    cv = tot[3 * NK]

    inv_b = jnp.float32(1.0 / B)
    m0 = [sum0[k] * inv_b for k in range(NK)]
    ilo8 = lax.iota(jnp.int32, L) < 8
    zero = jnp.zeros((L,), jnp.float32)
    m1 = []
    for k in range(NK):
        if k == 0:
            r = jnp.where(ilo8, sh[0], cv * m0[0])
        elif k == 1:
            r = jnp.where(ilo8, zero, cv * m0[1])
        else:
            r = cv * m0[k]
        m1.append((s1x[k] - sh[k] + r) * inv_b)
    rowz1 = jnp.where(ilo8, zero, m0[1])
    rowz2 = jnp.where(ilo8, zero, m1[1])

    base = wid * ROWS_PER_W
    pltpu.sync_copy(table0.at[pl.ds(base, ROWS_PER_W)],
                    fl0_v.at[pl.ds(0, ROWS_PER_W)])
    pltpu.sync_copy(table1.at[pl.ds(base, ROWS_PER_W)],
                    fl1_v.at[pl.ds(0, ROWS_PER_W)])
    lane = lax.iota(jnp.int32, L)

    def bld(g, cnt):
        f0 = fl0_v[pl.ds(g * L, L)]
        f1 = fl1_v[pl.ds(g * L, L)]
        m = (f0 > 0.0) | (f1 > 0.0)
        key = jnp.where(m, 0, 1).astype(jnp.int32)
        ids = jnp.full((L,), base + g * L, jnp.int32) + lane
        val = ((ids << 2)
               | jnp.where(f0 > 0.0, 2, 0).astype(jnp.int32)
               | jnp.where(f1 > 0.0, 1, 0).astype(jnp.int32))
        _, vs = plsc.sort_key_val(key, val)
        lst_v[pl.ds(cnt, L)] = vs
        return cnt + plsc.all_reduce_population_count(m)[0]
    cnt = lax.fori_loop(0, ROWS_PER_W // L, bld, jnp.int32(0))

    lastp = jnp.maximum(cnt - 1, 0)
    lid = jnp.full((L,), lst_v[pl.ds(lastp, L)][0], jnp.int32)
    for k in range(NK):
        lst_v[pl.ds(cnt + k * L, L)] = lid

    def chunk(ch, _):
        off = ch * CH
        for k in range(NK):
            idxw_v[0, pl.ds(k * L, L)] = lax.shift_right_logical(
                lst_v[pl.ds(off + k * L, L)], 2)
        pltpu.async_copy(x_hbm.at[idxw_v.at[0]], rows_v, sem).wait()

        def fix(i, _2):
            pv = lst_v[pl.ds(off + i, L)][0]
            g0 = (pv & 2) > 0
            g1 = (pv & 1) > 0
            v0 = rows_v[i, pl.ds(0, L)]
            p1 = jnp.where(ilo8, v0, m0[0])
            p2 = jnp.where(ilo8, m1[0], jnp.where(g0, m0[0], v0))
            rows_v[i, pl.ds(0, L)] = jnp.where(g1, p2, p1)
            rows_v[i, pl.ds(L, L)] = jnp.where(g1, rowz2, rowz1)
            for k in range(2, NK):
                rows_v[i, pl.ds(k * L, L)] = jnp.where(g1, m1[k], m0[k])
            return 0
        lax.fori_loop(0, CH, fix, 0)
        pltpu.async_copy(rows_v, out_hbm.at[idxw_v.at[0]], sem).wait()
        return 0
    trip = lax.shift_right_logical(cnt + (CH - 1), 7)
    lax.fori_loop(0, trip, chunk, 0)


def _make_kernel():
    mesh = plsc.VectorSubcoreMesh(core_axis_name="c", subcore_axis_name="s")
    return pl.kernel(
        _body,
        out_type=(),
        mesh=mesh,
        compiler_params=pltpu.CompilerParams(needs_layout_passes=False),
        scratch_types=[
            pltpu.VMEM((NCH, CH), jnp.int32),
            pltpu.VMEM((NCH, CH), jnp.int32),
            pltpu.VMEM((CH,), jnp.float32),
            pltpu.VMEM((CH, D), jnp.float32),
            pltpu.VMEM((CH + L,), jnp.float32),
            pltpu.VMEM((512,), jnp.float32),
            pltpu.VMEM((NS, 512), jnp.float32),
            pltpu.VMEM((N // NS,), jnp.float32),
            pltpu.VMEM((ROWS_PER_W + L,), jnp.float32),
            pltpu.VMEM((ROWS_PER_W + L,), jnp.float32),
            pltpu.VMEM((LSZ,), jnp.int32),
            pltpu.VMEM((1, CH), jnp.int32),
            pltpu.VMEM_SHARED((N,), jnp.float32),
            pltpu.VMEM_SHARED((N,), jnp.float32),
            pltpu.VMEM_SHARED((NS, 512), jnp.float32),
            pltpu.SemaphoreType.DMA,
        ],
    )


_sc_fixup = _make_kernel()


@jax.jit
def _selective_filter(x, idx0, idx1):
    out_ref = jax.new_ref(x)
    _sc_fixup(x, idx0, idx1, out_ref)
    return out_ref[...]


def kernel(input, idx0, idx1):
    return _selective_filter(input, idx0, idx1)
